# Initial kernel scaffold; baseline (speedup 1.0000x reference)
import jax
import jax.numpy as jnp
from jax.experimental import pallas as pl


def kernel(input_, weight):
    raise NotImplementedError("write your pallas kernel here")


# speedup vs baseline: 3.3561x; 3.3561x over previous
import functools

import jax
import jax.numpy as jnp
from jax import lax
from jax.experimental import pallas as pl
from jax.experimental.pallas import tpu as pltpu
from jax.experimental.pallas import tpu_sc as plsc

BATCH = 16384
HIST = 50
EMB = 128
B_TOTAL = BATCH * HIST
G = 128
NC = 2
NS = 16
NW = NC * NS
B_PER_W = B_TOTAL // NW
K = 2
CHUNK = K * G
N_CHUNKS = B_PER_W // CHUNK
GROUPS_PER_W = B_PER_W // G


@functools.partial(
    pl.kernel,
    out_type=jax.ShapeDtypeStruct((B_TOTAL, EMB), jnp.float32),
    mesh=plsc.VectorSubcoreMesh(core_axis_name="c", subcore_axis_name="s"),
    scratch_types=[
        pltpu.VMEM((2, K, G), jnp.int32),
        pltpu.VMEM((2, CHUNK, EMB), jnp.float32),
        pltpu.SemaphoreType.DMA,
        pltpu.SemaphoreType.DMA,
        pltpu.SemaphoreType.DMA,
    ],
)
def _gather_kernel(idx_hbm, table_hbm, out_hbm, idx_v, rows_v, gsem, osem0, osem1):
  wid = lax.axis_index("s") * NC + lax.axis_index("c")
  g0 = wid * GROUPS_PER_W
  base = wid * B_PER_W

  def pair(p, carry):
    for b in range(2):
      i = 2 * p + b
      osem = osem0 if b == 0 else osem1

      @pl.when(p > 0)
      def _():
        pltpu.make_async_copy(
            out_hbm.at[pl.ds(base, CHUNK)], rows_v.at[b], osem).wait()

      pltpu.sync_copy(idx_hbm.at[pl.ds(g0 + i * K, K)], idx_v.at[b])
      handles = [
          pltpu.async_copy(table_hbm.at[idx_v.at[b, j]],
                           rows_v.at[b, pl.ds(j * G, G)], gsem)
          for j in range(K)
      ]
      for h in handles:
        h.wait()
      pltpu.async_copy(
          rows_v.at[b], out_hbm.at[pl.ds(base + i * CHUNK, CHUNK)], osem)
    return carry

  lax.fori_loop(0, N_CHUNKS // 2, pair, 0)

  pltpu.make_async_copy(
      out_hbm.at[pl.ds(base, CHUNK)], rows_v.at[0], osem0).wait()
  pltpu.make_async_copy(
      out_hbm.at[pl.ds(base, CHUNK)], rows_v.at[1], osem1).wait()


def kernel(input_, weight):
  idx = input_.reshape(B_TOTAL // G, G).astype(jnp.int32)
  out = _gather_kernel(idx, weight)
  return out.reshape(BATCH, HIST, EMB)


# speedup vs baseline: 3.4581x; 1.0304x over previous
import functools

import jax
import jax.numpy as jnp
from jax import lax
from jax.experimental import pallas as pl
from jax.experimental.pallas import tpu as pltpu
from jax.experimental.pallas import tpu_sc as plsc

BATCH = 16384
HIST = 50
EMB = 128
B_TOTAL = BATCH * HIST
G = 128
NC = 2
NS = 16
NW = NC * NS
B_PER_W = B_TOTAL // NW
N_CHUNKS = B_PER_W // G
NBUF = 4
S_OUTER = N_CHUNKS // NBUF


@functools.partial(
    pl.kernel,
    out_type=jax.ShapeDtypeStruct((B_TOTAL, EMB), jnp.float32),
    mesh=plsc.VectorSubcoreMesh(core_axis_name="c", subcore_axis_name="s"),
    scratch_types=[
        pltpu.VMEM((N_CHUNKS, G), jnp.int32),
        pltpu.VMEM((NBUF, G, EMB), jnp.float32),
        pltpu.SemaphoreType.DMA((NBUF,)),
        pltpu.SemaphoreType.DMA((NBUF,)),
    ],
)
def _gather_kernel(idx_hbm, table_hbm, out_hbm, idx_v, rows_v, gsem, osem):
  wid = lax.axis_index("s") * NC + lax.axis_index("c")
  base = wid * B_PER_W

  def fire_gather(j, b):
    pltpu.async_copy(table_hbm.at[idx_v.at[j]], rows_v.at[b], gsem.at[b])

  def drain_out(b):
    pltpu.make_async_copy(
        out_hbm.at[pl.ds(base, G)], rows_v.at[b], osem.at[b]).wait()

  pltpu.sync_copy(idx_hbm.at[pl.ds(wid * N_CHUNKS, N_CHUNKS)], idx_v)

  for j in range(NBUF - 1):
    fire_gather(j, j)

  def outer(s, carry):
    for b in range(NBUF):
      i = s * NBUF + b
      pltpu.make_async_copy(
          table_hbm.at[idx_v.at[0]], rows_v.at[b], gsem.at[b]).wait()
      pltpu.async_copy(
          rows_v.at[b], out_hbm.at[pl.ds(base + i * G, G)], osem.at[b])
      bn = (b + NBUF - 1) % NBUF
      if b == 0:
        @pl.when(s > 0)
        def _():
          drain_out(bn)
        fire_gather(i + NBUF - 1, bn)
      else:
        @pl.when(s < S_OUTER - 1)
        def _():
          drain_out(bn)
          fire_gather(i + NBUF - 1, bn)
    return carry

  lax.fori_loop(0, S_OUTER, outer, 0)

  for b in range(NBUF):
    drain_out(b)


def kernel(input_, weight):
  idx = input_.reshape(B_TOTAL // G, G).astype(jnp.int32)
  out = _gather_kernel(idx, weight)
  return out.reshape(BATCH, HIST, EMB)


# speedup vs baseline: 6.3888x; 1.8475x over previous
import functools

import jax
import jax.numpy as jnp
from jax import lax
from jax.experimental import pallas as pl
from jax.experimental.pallas import tpu as pltpu
from jax.experimental.pallas import tpu_sc as plsc

BATCH = 16384
HIST = 50
EMB = 128
NC = 2
NS = 16
NW = NC * NS
BATCH_PER_W = BATCH // NW
CB = 2
N_CHUNKS = BATCH_PER_W // CB
NBUF = 4
S_OUTER = N_CHUNKS // NBUF


@functools.partial(
    pl.kernel,
    out_type=jax.ShapeDtypeStruct((BATCH, HIST, EMB), jnp.float32),
    mesh=plsc.VectorSubcoreMesh(core_axis_name="c", subcore_axis_name="s"),
    scratch_types=[
        pltpu.VMEM((BATCH_PER_W, HIST), jnp.int32),
        pltpu.VMEM((NBUF, CB, HIST, EMB), jnp.float32),
        pltpu.SemaphoreType.DMA((NBUF,)),
        pltpu.SemaphoreType.DMA((NBUF,)),
    ],
)
def _gather_kernel(idx_hbm, table_hbm, out_hbm, idx_v, rows_v, gsem, osem):
  wid = lax.axis_index("s") * NC + lax.axis_index("c")
  batch0 = wid * BATCH_PER_W

  def fire_gather(c, b):
    for j in range(CB):
      pltpu.async_copy(table_hbm.at[idx_v.at[CB * c + j]],
                       rows_v.at[b, j], gsem.at[b])

  def wait_gather(b):
    pltpu.make_async_copy(
        out_hbm.at[pl.ds(0, CB)], rows_v.at[b], gsem.at[b]).wait()

  def fire_out(c, b):
    pltpu.async_copy(
        rows_v.at[b], out_hbm.at[pl.ds(batch0 + c * CB, CB)], osem.at[b])

  def drain_out(b):
    pltpu.make_async_copy(
        out_hbm.at[pl.ds(0, CB)], rows_v.at[b], osem.at[b]).wait()

  pltpu.sync_copy(idx_hbm.at[pl.ds(batch0, BATCH_PER_W)], idx_v)

  for c in range(NBUF - 1):
    fire_gather(c, c)

  def outer(s, carry):
    for b in range(NBUF):
      c = s * NBUF + b
      wait_gather(b)
      fire_out(c, b)
      bn = (b + NBUF - 1) % NBUF
      if b == 0:
        @pl.when(s > 0)
        def _():
          drain_out(bn)
        fire_gather(c + NBUF - 1, bn)
      else:
        @pl.when(s < S_OUTER - 1)
        def _():
          drain_out(bn)
          fire_gather(c + NBUF - 1, bn)
    return carry

  lax.fori_loop(0, S_OUTER, outer, 0)

  for b in range(NBUF):
    drain_out(b)


def kernel(input_, weight):
  return _gather_kernel(input_.astype(jnp.int32), weight)


# speedup vs baseline: 11.9835x; 1.8757x over previous
import functools

import jax
import jax.numpy as jnp
from jax import lax
from jax.experimental import pallas as pl
from jax.experimental.pallas import tpu as pltpu
from jax.experimental.pallas import tpu_sc as plsc

BATCH = 16384
HIST = 50
EMB = 128
B_TOTAL = BATCH * HIST
G = 128
NC = 2
NS = 16
NW = NC * NS
B_PER_W = B_TOTAL // NW
N_CHUNKS = B_PER_W // G
NBUF = 4
S_OUTER = N_CHUNKS // NBUF


@functools.partial(
    pl.kernel,
    out_type=jax.ShapeDtypeStruct((B_TOTAL, EMB), jnp.float32),
    mesh=plsc.VectorSubcoreMesh(core_axis_name="c", subcore_axis_name="s"),
    scratch_types=[
        pltpu.VMEM((N_CHUNKS, G), jnp.int32),
        pltpu.VMEM((NBUF, G, EMB), jnp.float32),
        pltpu.SemaphoreType.DMA((NBUF,)),
        pltpu.SemaphoreType.DMA((NBUF,)),
    ],
)
def _gather_kernel(idx_hbm, table_hbm, out_hbm, idx_v, rows_v, gsem, osem):
  wid = lax.axis_index("s") * NC + lax.axis_index("c")
  base = wid * B_PER_W

  def fire_gather(j, b):
    pltpu.async_copy(table_hbm.at[idx_v.at[j]], rows_v.at[b], gsem.at[b])

  def drain_out(b):
    pltpu.make_async_copy(
        out_hbm.at[pl.ds(base, G)], rows_v.at[b], osem.at[b]).wait()

  pltpu.sync_copy(idx_hbm.at[pl.ds(wid * N_CHUNKS, N_CHUNKS)], idx_v)

  for j in range(NBUF - 1):
    fire_gather(j, j)

  def outer(s, carry):
    for b in range(NBUF):
      i = s * NBUF + b
      pltpu.make_async_copy(
          table_hbm.at[idx_v.at[0]], rows_v.at[b], gsem.at[b]).wait()
      pltpu.async_copy(
          rows_v.at[b], out_hbm.at[pl.ds(base + i * G, G)], osem.at[b])
      bn = (b + NBUF - 1) % NBUF
      if b == 0:
        @pl.when(s > 0)
        def _():
          drain_out(bn)
        fire_gather(i + NBUF - 1, bn)
      else:
        @pl.when(s < S_OUTER - 1)
        def _():
          drain_out(bn)
          fire_gather(i + NBUF - 1, bn)
    return carry

  lax.fori_loop(0, S_OUTER, outer, 0)

  for b in range(NBUF):
    drain_out(b)


def kernel(input_, weight):
  idx = input_.T.reshape(B_TOTAL // G, G).astype(jnp.int32)
  out = _gather_kernel(idx, weight)
  return out.reshape(HIST, BATCH, EMB).transpose(1, 0, 2)
